# baseline (speedup 1.0000x reference)
import functools

import jax
import jax.numpy as jnp
from jax import lax
from jax.experimental import pallas as pl
from jax.experimental.pallas import tpu as pltpu
from jax.experimental.pallas import tpu_sc as plsc

NC = 2
NS = 16
NW = NC * NS
G = 96


def _tc_prep(x, W, A2, BN):
    N, D = x.shape
    K = A2.shape[1]

    def body(x_ref, w_ref, a2_ref, h_ref, t_ref):
        h = jnp.dot(x_ref[...], w_ref[...], preferred_element_type=jnp.float32)
        h_ref[...] = h
        t_ref[...] = jnp.dot(h, a2_ref[...], preferred_element_type=jnp.float32)

    return pl.pallas_call(
        body,
        grid=(N // BN,),
        in_specs=[
            pl.BlockSpec((BN, D), lambda i: (i, 0)),
            pl.BlockSpec((D, D), lambda i: (0, 0)),
            pl.BlockSpec((D, K), lambda i: (0, 0)),
        ],
        out_specs=[
            pl.BlockSpec((BN, D), lambda i: (i, 0)),
            pl.BlockSpec((BN, K), lambda i: (i, 0)),
        ],
        out_shape=[
            jax.ShapeDtypeStruct((N, D), jnp.float32),
            jax.ShapeDtypeStruct((N, K), jnp.float32),
        ],
    )(x, W, A2)


def _tc_finish(parts, dens, Bsel, bias2d, BN):
    _, N, D = parts.shape

    def body(p_ref, d_ref, b_ref, bias_ref, o_ref):
        acc = p_ref[0] + p_ref[1]
        den = d_ref[0] + d_ref[1]
        dex = jnp.dot(den, b_ref[...], preferred_element_type=jnp.float32)
        o_ref[...] = jnp.maximum(acc / (dex + 1e-16) + bias_ref[...], 0.0)

    return pl.pallas_call(
        body,
        grid=(N // BN,),
        in_specs=[
            pl.BlockSpec((2, BN, D), lambda i: (0, i, 0)),
            pl.BlockSpec((2, BN, 16), lambda i: (0, i, 0)),
            pl.BlockSpec((16, D), lambda i: (0, 0)),
            pl.BlockSpec((1, D), lambda i: (0, 0)),
        ],
        out_specs=pl.BlockSpec((BN, D), lambda i: (i, 0)),
        out_shape=jax.ShapeDtypeStruct((N, D), jnp.float32),
    )(parts, dens, Bsel, bias2d)


def _edge_pass(ei3, s_tab, d_tab, h, z16, zD):
    NP, D = zD.shape[0], zD.shape[1]
    NROWS = ei3.shape[0]
    mesh = plsc.VectorSubcoreMesh(core_axis_name="c", subcore_axis_name="s")
    rpw = NP // NS
    gps = NROWS // NW
    HB = D // 16

    data_set = [
        pltpu.VMEM((G, 16), jnp.float32),
        pltpu.VMEM((G, 16), jnp.float32),
        pltpu.VMEM((G, D), jnp.float32),
        pltpu.SemaphoreType.DMA,
    ]
    idx_set = [
        pltpu.VMEM((2, G), jnp.int32),
        pltpu.SemaphoreType.DMA,
    ]

    @functools.partial(
        pl.kernel,
        out_type=[
            jax.ShapeDtypeStruct((NC, NP, D), jnp.float32),
            jax.ShapeDtypeStruct((NC, NP, 16), jnp.float32),
        ],
        mesh=mesh,
        scratch_types=data_set * 2 + idx_set * 4 + [
            pltpu.VMEM_SHARED((NP, D), jnp.float32),
            pltpu.VMEM_SHARED((NP, 16), jnp.float32),
        ],
        compiler_params=pltpu.CompilerParams(use_tc_tiling_on_sc=False),
    )
    def kern(ei_hbm, stab_hbm, dtab_hbm, h_hbm, z16_hbm, zD_hbm,
             outp_hbm, dparts_hbm, *scratch):
        dsets = [scratch[i * 4:(i + 1) * 4] for i in range(2)]
        isets = [scratch[8 + i * 2: 8 + (i + 1) * 2] for i in range(4)]
        out_sh, den_sh = scratch[16], scratch[17]
        c = lax.axis_index("c")
        s = lax.axis_index("s")
        wid = c * NS + s
        pltpu.sync_copy(zD_hbm.at[pl.ds(s * rpw, rpw)],
                        out_sh.at[pl.ds(s * rpw, rpw)])
        pltpu.sync_copy(z16_hbm.at[pl.ds(s * rpw, rpw)],
                        den_sh.at[pl.ds(s * rpw, rpw)])
        plsc.subcore_barrier()

        row0 = wid * gps
        last = row0 + gps - 1

        def issue_idx(row, it):
            pltpu.async_copy(ei_hbm.at[row], it[0], it[1])

        def wait_idx(it):
            pltpu.make_async_copy(ei_hbm.at[0], it[0], it[1]).wait()

        def issue_gathers(dt, it):
            pltpu.async_copy(stab_hbm.at[it[0].at[0]], dt[0], dt[3])
            pltpu.async_copy(dtab_hbm.at[it[0].at[1]], dt[1], dt[3])
            pltpu.async_copy(h_hbm.at[it[0].at[0]], dt[2], dt[3])

        def wait_gathers(dt, it):
            pltpu.make_async_copy(stab_hbm.at[it[0].at[0]], dt[0], dt[3]).wait()
            pltpu.make_async_copy(dtab_hbm.at[it[0].at[1]], dt[1], dt[3]).wait()
            pltpu.make_async_copy(h_hbm.at[it[0].at[0]], dt[2], dt[3]).wait()

        def compute(dt, it):
            srow, drow, hrows = dt[0], dt[1], dt[2]

            def cbody(e, carry2):
                v = srow[e, :] + drow[e, :]
                v = jnp.maximum(v, 0.2 * v)
                pv = jnp.exp(v)
                srow[e, :] = pv
                for hb in range(HB):
                    cs = pv[hb]
                    hrows[e, pl.ds(hb * 16, 16)] = hrows[e, pl.ds(hb * 16, 16)] * cs
                return carry2

            lax.fori_loop(0, G, cbody, 0)
            pltpu.sync_copy(hrows, out_sh.at[it[0].at[1]], add=True)
            pltpu.sync_copy(srow, den_sh.at[it[0].at[1]], add=True)

        issue_idx(row0, isets[0])
        issue_idx(row0 + 1, isets[1])
        wait_idx(isets[0])
        issue_gathers(dsets[0], isets[0])
        issue_idx(row0 + 2, isets[2])

        def body(j, carry):
            g0 = row0 + 4 * j
            for p in range(4):
                wait_gathers(dsets[p % 2], isets[p])
                wait_idx(isets[(p + 1) % 4])
                issue_gathers(dsets[(p + 1) % 2], isets[(p + 1) % 4])
                issue_idx(jnp.minimum(g0 + p + 3, last), isets[(p + 3) % 4])
                compute(dsets[p % 2], isets[p])
            return carry

        lax.fori_loop(0, gps // 4, body, 0)
        wait_gathers(dsets[0], isets[0])
        wait_idx(isets[1])
        wait_idx(isets[2])
        plsc.subcore_barrier()
        pltpu.sync_copy(out_sh.at[pl.ds(s * rpw, rpw)],
                        outp_hbm.at[c, pl.ds(s * rpw, rpw)])
        pltpu.sync_copy(den_sh.at[pl.ds(s * rpw, rpw)],
                        dparts_hbm.at[c, pl.ds(s * rpw, rpw)])

    return kern(ei3, s_tab, d_tab, h, z16, zD)


def kernel(x, edge_index, W, att_src, att_dst, bias):
    N, D = x.shape
    E = edge_index.shape[1]
    H, C = att_src.shape

    eye = jnp.eye(H, dtype=jnp.float32)
    Asrc = (att_src[:, :, None] * eye[:, None, :]).reshape(H * C, H)
    Adst = (att_dst[:, :, None] * eye[:, None, :]).reshape(H * C, H)
    A2 = jnp.concatenate([Asrc, Asrc, Adst, Adst], axis=1)

    NP = ((N + 2047) // 2048) * 2048
    xp = jnp.concatenate([x, jnp.zeros((NP - N, D), jnp.float32)], axis=0)
    h, t = _tc_prep(xp, W, A2, BN=1024)
    s_tab = t[:, :16]
    d_tab = t[:, 16:]

    NG = ((E + NW * G * 4 - 1) // (NW * G * 4)) * NW * G * 4
    pad = jnp.full((2, NG - E), NP - 1, jnp.int32)
    ei3 = (jnp.concatenate([edge_index, pad], axis=1)
           .reshape(2, NG // G, G).transpose(1, 0, 2))
    z16 = jnp.zeros((NP, 16), jnp.float32)
    zD = jnp.zeros((NP, D), jnp.float32)

    parts, dens = _edge_pass(ei3, s_tab, d_tab, h, z16, zD)

    l = jnp.arange(D)
    Bsel = (jnp.arange(16)[:, None] == (l[None, :] // C)).astype(jnp.float32)
    out = _tc_finish(parts, dens, Bsel, bias.reshape(1, D), BN=1024)
    return out[:N]


# speedup vs baseline: 1.0905x; 1.0905x over previous
import functools

import jax
import jax.numpy as jnp
from jax import lax
from jax.experimental import pallas as pl
from jax.experimental.pallas import tpu as pltpu
from jax.experimental.pallas import tpu_sc as plsc

NC = 2
NS = 16
NW = NC * NS
G = 128


def _tc_prep(x, W, A2, BN):
    N, D = x.shape
    K = A2.shape[1]

    def body(x_ref, w_ref, a2_ref, h_ref, t_ref):
        h = jnp.dot(x_ref[...], w_ref[...], preferred_element_type=jnp.float32)
        h_ref[...] = h
        t_ref[...] = jnp.dot(h, a2_ref[...], preferred_element_type=jnp.float32)

    return pl.pallas_call(
        body,
        grid=(N // BN,),
        in_specs=[
            pl.BlockSpec((BN, D), lambda i: (i, 0)),
            pl.BlockSpec((D, D), lambda i: (0, 0)),
            pl.BlockSpec((D, K), lambda i: (0, 0)),
        ],
        out_specs=[
            pl.BlockSpec((BN, D), lambda i: (i, 0)),
            pl.BlockSpec((BN, K), lambda i: (i, 0)),
        ],
        out_shape=[
            jax.ShapeDtypeStruct((N, D), jnp.float32),
            jax.ShapeDtypeStruct((N, K), jnp.float32),
        ],
    )(x, W, A2)


def _tc_finish(parts, dens, Bsel, bias2d, BN):
    _, N, D = parts.shape

    def body(p_ref, d_ref, b_ref, bias_ref, o_ref):
        acc = p_ref[0] + p_ref[1]
        den = d_ref[0] + d_ref[1]
        dex = jnp.dot(den, b_ref[...], preferred_element_type=jnp.float32)
        o_ref[...] = jnp.maximum(acc / (dex + 1e-16) + bias_ref[...], 0.0)

    return pl.pallas_call(
        body,
        grid=(N // BN,),
        in_specs=[
            pl.BlockSpec((2, BN, D), lambda i: (0, i, 0)),
            pl.BlockSpec((2, BN, 16), lambda i: (0, i, 0)),
            pl.BlockSpec((16, D), lambda i: (0, 0)),
            pl.BlockSpec((1, D), lambda i: (0, 0)),
        ],
        out_specs=pl.BlockSpec((BN, D), lambda i: (i, 0)),
        out_shape=jax.ShapeDtypeStruct((N, D), jnp.float32),
    )(parts, dens, Bsel, bias2d)


def _edge_pass_a(ei3, s_tab, d_tab, z16):
    NP = z16.shape[0]
    NROWS = ei3.shape[0]
    mesh = plsc.VectorSubcoreMesh(core_axis_name="c", subcore_axis_name="s")
    rpw = NP // NS
    gps = NROWS // NW

    data_set = [
        pltpu.VMEM((G, 16), jnp.float32),
        pltpu.VMEM((G, 16), jnp.float32),
        pltpu.SemaphoreType.DMA,
    ]
    idx_set = [
        pltpu.VMEM((2, G), jnp.int32),
        pltpu.SemaphoreType.DMA,
    ]

    @functools.partial(
        pl.kernel,
        out_type=[
            jax.ShapeDtypeStruct((NROWS, G, 16), jnp.float32),
            jax.ShapeDtypeStruct((NC, NP, 16), jnp.float32),
        ],
        mesh=mesh,
        scratch_types=data_set * 2 + idx_set * 4 + [
            pltpu.VMEM_SHARED((NP, 16), jnp.float32),
        ],
        compiler_params=pltpu.CompilerParams(use_tc_tiling_on_sc=False),
    )
    def kern(ei_hbm, stab_hbm, dtab_hbm, z16_hbm, p_hbm, dparts_hbm, *scratch):
        dsets = [scratch[i * 3:(i + 1) * 3] for i in range(2)]
        isets = [scratch[6 + i * 2: 6 + (i + 1) * 2] for i in range(4)]
        den_sh = scratch[14]
        c = lax.axis_index("c")
        s = lax.axis_index("s")
        wid = c * NS + s
        pltpu.sync_copy(z16_hbm.at[pl.ds(s * rpw, rpw)],
                        den_sh.at[pl.ds(s * rpw, rpw)])
        plsc.subcore_barrier()

        row0 = wid * gps
        last = row0 + gps - 1

        def issue_idx(row, it):
            pltpu.async_copy(ei_hbm.at[row], it[0], it[1])

        def wait_idx(it):
            pltpu.make_async_copy(ei_hbm.at[0], it[0], it[1]).wait()

        def issue_gathers(dt, it):
            pltpu.async_copy(stab_hbm.at[it[0].at[0]], dt[0], dt[2])
            pltpu.async_copy(dtab_hbm.at[it[0].at[1]], dt[1], dt[2])

        def wait_gathers(dt, it):
            pltpu.make_async_copy(stab_hbm.at[it[0].at[0]], dt[0], dt[2]).wait()
            pltpu.make_async_copy(dtab_hbm.at[it[0].at[1]], dt[1], dt[2]).wait()

        def compute(row, dt, it):
            srow, drow = dt[0], dt[1]

            def cbody(e, carry2):
                v = srow[e, :] + drow[e, :]
                v = jnp.maximum(v, 0.2 * v)
                srow[e, :] = jnp.exp(v)
                return carry2

            lax.fori_loop(0, G, cbody, 0)
            pltpu.sync_copy(srow, p_hbm.at[row])
            pltpu.sync_copy(srow, den_sh.at[it[0].at[1]], add=True)

        issue_idx(row0, isets[0])
        issue_idx(row0 + 1, isets[1])
        wait_idx(isets[0])
        issue_gathers(dsets[0], isets[0])
        issue_idx(row0 + 2, isets[2])

        def body(j, carry):
            g0 = row0 + 4 * j
            for p in range(4):
                wait_gathers(dsets[p % 2], isets[p])
                wait_idx(isets[(p + 1) % 4])
                issue_gathers(dsets[(p + 1) % 2], isets[(p + 1) % 4])
                issue_idx(jnp.minimum(g0 + p + 3, last), isets[(p + 3) % 4])
                compute(g0 + p, dsets[p % 2], isets[p])
            return carry

        lax.fori_loop(0, gps // 4, body, 0)
        wait_gathers(dsets[0], isets[0])
        wait_idx(isets[1])
        wait_idx(isets[2])
        plsc.subcore_barrier()
        pltpu.sync_copy(den_sh.at[pl.ds(s * rpw, rpw)],
                        dparts_hbm.at[c, pl.ds(s * rpw, rpw)])

    return kern(ei3, s_tab, d_tab, z16)


def _edge_pass_b(ei3, p3d, h, zD):
    NP, D = zD.shape[0], zD.shape[1]
    NROWS = ei3.shape[0]
    mesh = plsc.VectorSubcoreMesh(core_axis_name="c", subcore_axis_name="s")
    rpw = NP // NS
    gps = NROWS // NW
    HB = D // 16

    data_set = [
        pltpu.VMEM((G, 16), jnp.float32),
        pltpu.VMEM((G, D), jnp.float32),
        pltpu.SemaphoreType.DMA,
    ]
    idx_set = [
        pltpu.VMEM((2, G), jnp.int32),
        pltpu.SemaphoreType.DMA,
    ]

    @functools.partial(
        pl.kernel,
        out_type=jax.ShapeDtypeStruct((NC, NP, D), jnp.float32),
        mesh=mesh,
        scratch_types=data_set * 2 + idx_set * 4 + [
            pltpu.VMEM_SHARED((NP, D), jnp.float32),
        ],
        compiler_params=pltpu.CompilerParams(use_tc_tiling_on_sc=False),
    )
    def kern(ei_hbm, p_hbm, h_hbm, zD_hbm, outp_hbm, *scratch):
        dsets = [scratch[i * 3:(i + 1) * 3] for i in range(2)]
        isets = [scratch[6 + i * 2: 6 + (i + 1) * 2] for i in range(4)]
        out_sh = scratch[14]
        c = lax.axis_index("c")
        s = lax.axis_index("s")
        wid = c * NS + s
        pltpu.sync_copy(zD_hbm.at[pl.ds(s * rpw, rpw)],
                        out_sh.at[pl.ds(s * rpw, rpw)])
        plsc.subcore_barrier()

        row0 = wid * gps
        last = row0 + gps - 1

        def issue_idx(row, it):
            pltpu.async_copy(ei_hbm.at[row], it[0], it[1])

        def wait_idx(it):
            pltpu.make_async_copy(ei_hbm.at[0], it[0], it[1]).wait()

        def issue_gathers(row, dt, it):
            pltpu.async_copy(p_hbm.at[row], dt[0], dt[2])
            pltpu.async_copy(h_hbm.at[it[0].at[0]], dt[1], dt[2])

        def wait_gathers(dt, it):
            pltpu.make_async_copy(p_hbm.at[0], dt[0], dt[2]).wait()
            pltpu.make_async_copy(h_hbm.at[it[0].at[0]], dt[1], dt[2]).wait()

        def compute(dt, it):
            prow, hrows = dt[0], dt[1]

            def mbody(e, carry2):
                pv = prow[e, :]
                for hb in range(HB):
                    cs = pv[hb]
                    hrows[e, pl.ds(hb * 16, 16)] = hrows[e, pl.ds(hb * 16, 16)] * cs
                return carry2

            lax.fori_loop(0, G, mbody, 0)
            pltpu.sync_copy(hrows, out_sh.at[it[0].at[1]], add=True)

        issue_idx(row0, isets[0])
        issue_idx(row0 + 1, isets[1])
        wait_idx(isets[0])
        issue_gathers(row0, dsets[0], isets[0])
        issue_idx(row0 + 2, isets[2])

        def body(j, carry):
            g0 = row0 + 4 * j
            for p in range(4):
                wait_gathers(dsets[p % 2], isets[p])
                wait_idx(isets[(p + 1) % 4])
                issue_gathers(jnp.minimum(g0 + p + 1, last),
                              dsets[(p + 1) % 2], isets[(p + 1) % 4])
                issue_idx(jnp.minimum(g0 + p + 3, last), isets[(p + 3) % 4])
                compute(dsets[p % 2], isets[p])
            return carry

        lax.fori_loop(0, gps // 4, body, 0)
        wait_gathers(dsets[0], isets[0])
        wait_idx(isets[1])
        wait_idx(isets[2])
        plsc.subcore_barrier()
        pltpu.sync_copy(out_sh.at[pl.ds(s * rpw, rpw)],
                        outp_hbm.at[c, pl.ds(s * rpw, rpw)])

    return kern(ei3, p3d, h, zD)


def kernel(x, edge_index, W, att_src, att_dst, bias):
    N, D = x.shape
    E = edge_index.shape[1]
    H, C = att_src.shape

    eye = jnp.eye(H, dtype=jnp.float32)
    Asrc = (att_src[:, :, None] * eye[:, None, :]).reshape(H * C, H)
    Adst = (att_dst[:, :, None] * eye[:, None, :]).reshape(H * C, H)
    A2 = jnp.concatenate([Asrc, Asrc, Adst, Adst], axis=1)

    NP = ((N + 2047) // 2048) * 2048
    xp = jnp.concatenate([x, jnp.zeros((NP - N, D), jnp.float32)], axis=0)
    h, t = _tc_prep(xp, W, A2, BN=1024)
    s_tab = t[:, :16]
    d_tab = t[:, 16:]

    NG = ((E + NW * G * 4 - 1) // (NW * G * 4)) * NW * G * 4
    pad = jnp.full((2, NG - E), NP - 1, jnp.int32)
    ei3 = (jnp.concatenate([edge_index, pad], axis=1)
           .reshape(2, NG // G, G).transpose(1, 0, 2))
    z16 = jnp.zeros((NP, 16), jnp.float32)
    zD = jnp.zeros((NP, D), jnp.float32)

    p3d, dens = _edge_pass_a(ei3, s_tab, d_tab, z16)
    parts = _edge_pass_b(ei3, p3d, h, zD)

    l = jnp.arange(D)
    Bsel = (jnp.arange(16)[:, None] == (l[None, :] // C)).astype(jnp.float32)
    out = _tc_finish(parts, dens, Bsel, bias.reshape(1, D), BN=1024)
    return out[:N]


# speedup vs baseline: 2.0007x; 1.8346x over previous
import functools

import jax
import jax.numpy as jnp
from jax import lax
from jax.experimental import pallas as pl
from jax.experimental.pallas import tpu as pltpu
from jax.experimental.pallas import tpu_sc as plsc

NC = 2
NS = 16
NW = NC * NS
G = 128


def _tc_prep(x, W, A2, BN):
    N, D = x.shape
    K = A2.shape[1]

    def body(x_ref, w_ref, a2_ref, h_ref, t_ref):
        h = jnp.dot(x_ref[...], w_ref[...], preferred_element_type=jnp.float32)
        h_ref[...] = h
        t_ref[...] = jnp.dot(h, a2_ref[...], preferred_element_type=jnp.float32)

    return pl.pallas_call(
        body,
        grid=(N // BN,),
        in_specs=[
            pl.BlockSpec((BN, D), lambda i: (i, 0)),
            pl.BlockSpec((D, D), lambda i: (0, 0)),
            pl.BlockSpec((D, K), lambda i: (0, 0)),
        ],
        out_specs=[
            pl.BlockSpec((BN, D), lambda i: (i, 0)),
            pl.BlockSpec((BN, K), lambda i: (i, 0)),
        ],
        out_shape=[
            jax.ShapeDtypeStruct((N, D), jnp.float32),
            jax.ShapeDtypeStruct((N, K), jnp.float32),
        ],
    )(x, W, A2)


def _tc_finish(parts, dens, Bsel, bias2d, BN):
    _, N, D = parts.shape

    def body(p_ref, d_ref, b_ref, bias_ref, o_ref):
        acc = p_ref[0] + p_ref[1]
        den = d_ref[0] + d_ref[1]
        dex = jnp.dot(den, b_ref[...], preferred_element_type=jnp.float32)
        o_ref[...] = jnp.maximum(acc / (dex + 1e-16) + bias_ref[...], 0.0)

    return pl.pallas_call(
        body,
        grid=(N // BN,),
        in_specs=[
            pl.BlockSpec((2, BN, D), lambda i: (0, i, 0)),
            pl.BlockSpec((2, BN, 16), lambda i: (0, i, 0)),
            pl.BlockSpec((16, D), lambda i: (0, 0)),
            pl.BlockSpec((1, D), lambda i: (0, 0)),
        ],
        out_specs=pl.BlockSpec((BN, D), lambda i: (i, 0)),
        out_shape=jax.ShapeDtypeStruct((N, D), jnp.float32),
    )(parts, dens, Bsel, bias2d)


def _edge_pass_a(ei3, s_tab, d_tab, z16):
    NP = z16.shape[0]
    NROWS = ei3.shape[0]
    mesh = plsc.VectorSubcoreMesh(core_axis_name="c", subcore_axis_name="s")
    rpw = NP // NS
    gps = NROWS // NW

    data_set = [
        pltpu.VMEM((G, 16), jnp.float32),
        pltpu.VMEM((G, 16), jnp.float32),
        pltpu.SemaphoreType.DMA,
    ]
    idx_set = [
        pltpu.VMEM((2, G), jnp.int32),
        pltpu.SemaphoreType.DMA,
    ]

    @functools.partial(
        pl.kernel,
        out_type=[
            jax.ShapeDtypeStruct((NROWS, G, 16), jnp.float32),
            jax.ShapeDtypeStruct((NC, NP, 16), jnp.float32),
        ],
        mesh=mesh,
        scratch_types=data_set * 2 + idx_set * 4 + [
            pltpu.VMEM_SHARED((NP, 16), jnp.float32),
        ],
        compiler_params=pltpu.CompilerParams(use_tc_tiling_on_sc=False),
    )
    def kern(ei_hbm, stab_hbm, dtab_hbm, z16_hbm, p_hbm, dparts_hbm, *scratch):
        dsets = [scratch[i * 3:(i + 1) * 3] for i in range(2)]
        isets = [scratch[6 + i * 2: 6 + (i + 1) * 2] for i in range(4)]
        den_sh = scratch[14]
        c = lax.axis_index("c")
        s = lax.axis_index("s")
        wid = c * NS + s
        pltpu.sync_copy(z16_hbm.at[pl.ds(s * rpw, rpw)],
                        den_sh.at[pl.ds(s * rpw, rpw)])
        plsc.subcore_barrier()

        row0 = wid * gps
        last = row0 + gps - 1

        def issue_idx(row, it):
            pltpu.async_copy(ei_hbm.at[row], it[0], it[1])

        def wait_idx(it):
            pltpu.make_async_copy(ei_hbm.at[0], it[0], it[1]).wait()

        def issue_gathers(dt, it):
            pltpu.async_copy(stab_hbm.at[it[0].at[0]], dt[0], dt[2])
            pltpu.async_copy(dtab_hbm.at[it[0].at[1]], dt[1], dt[2])

        def wait_gathers(dt, it):
            pltpu.make_async_copy(stab_hbm.at[it[0].at[0]], dt[0], dt[2]).wait()
            pltpu.make_async_copy(dtab_hbm.at[it[0].at[1]], dt[1], dt[2]).wait()

        def compute(row, dt, it):
            srow, drow = dt[0], dt[1]

            def cbody(e, carry2):
                v = srow[e, :] + drow[e, :]
                v = jnp.maximum(v, 0.2 * v)
                srow[e, :] = jnp.exp(v)
                return carry2

            lax.fori_loop(0, G, cbody, 0)
            pltpu.sync_copy(srow, p_hbm.at[row])
            pltpu.sync_copy(srow, den_sh.at[it[0].at[1]], add=True)

        issue_idx(row0, isets[0])
        issue_idx(row0 + 1, isets[1])
        wait_idx(isets[0])
        issue_gathers(dsets[0], isets[0])
        issue_idx(row0 + 2, isets[2])

        def body(j, carry):
            g0 = row0 + 4 * j
            for p in range(4):
                wait_gathers(dsets[p % 2], isets[p])
                wait_idx(isets[(p + 1) % 4])
                issue_gathers(dsets[(p + 1) % 2], isets[(p + 1) % 4])
                issue_idx(jnp.minimum(g0 + p + 3, last), isets[(p + 3) % 4])
                compute(g0 + p, dsets[p % 2], isets[p])
            return carry

        lax.fori_loop(0, gps // 4, body, 0)
        wait_gathers(dsets[0], isets[0])
        wait_idx(isets[1])
        wait_idx(isets[2])
        plsc.subcore_barrier()
        pltpu.sync_copy(den_sh.at[pl.ds(s * rpw, rpw)],
                        dparts_hbm.at[c, pl.ds(s * rpw, rpw)])

    return kern(ei3, s_tab, d_tab, z16)


def _edge_pass_b(ei3, p3d, h, zD):
    NP, D = zD.shape[0], zD.shape[1]
    NROWS = ei3.shape[0]
    mesh = plsc.VectorSubcoreMesh(core_axis_name="c", subcore_axis_name="s")
    rpw = NP // NS
    gps = NROWS // NW
    HB = D // 16

    data_set = [
        pltpu.VMEM((G, 16), jnp.float32),
        pltpu.VMEM((G, D), jnp.float32),
        pltpu.SemaphoreType.DMA,
    ]
    idx_set = [
        pltpu.VMEM((2, G), jnp.int32),
        pltpu.SemaphoreType.DMA,
    ]

    @functools.partial(
        pl.kernel,
        out_type=jax.ShapeDtypeStruct((NC, NP, D), jnp.float32),
        mesh=mesh,
        scratch_types=data_set * 2 + idx_set * 4 + [
            pltpu.VMEM_SHARED((NP, D), jnp.float32),
        ],
        compiler_params=pltpu.CompilerParams(use_tc_tiling_on_sc=False),
    )
    def kern(ei_hbm, p_hbm, h_hbm, zD_hbm, outp_hbm, *scratch):
        dsets = [scratch[i * 3:(i + 1) * 3] for i in range(2)]
        isets = [scratch[6 + i * 2: 6 + (i + 1) * 2] for i in range(4)]
        out_sh = scratch[14]
        c = lax.axis_index("c")
        s = lax.axis_index("s")
        wid = c * NS + s
        pltpu.sync_copy(zD_hbm.at[pl.ds(s * rpw, rpw)],
                        out_sh.at[pl.ds(s * rpw, rpw)])
        plsc.subcore_barrier()

        row0 = wid * gps
        last = row0 + gps - 1

        def issue_idx(row, it):
            pltpu.async_copy(ei_hbm.at[row], it[0], it[1])

        def wait_idx(it):
            pltpu.make_async_copy(ei_hbm.at[0], it[0], it[1]).wait()

        def issue_gathers(row, dt, it):
            pltpu.async_copy(p_hbm.at[row], dt[0], dt[2])
            pltpu.async_copy(h_hbm.at[it[0].at[0]], dt[1], dt[2])

        def wait_gathers(dt, it):
            pltpu.make_async_copy(p_hbm.at[0], dt[0], dt[2]).wait()
            pltpu.make_async_copy(h_hbm.at[it[0].at[0]], dt[1], dt[2]).wait()

        def compute(dt, it):
            prow, hrows = dt[0], dt[1]

            def mbody(e, carry2):
                pv = prow[e, :]
                for hb in range(HB):
                    cs = pv[hb]
                    hrows[e, pl.ds(hb * 16, 16)] = hrows[e, pl.ds(hb * 16, 16)] * cs
                return carry2

            lax.fori_loop(0, G, mbody, 0)
            pltpu.sync_copy(hrows, out_sh.at[it[0].at[1]], add=True)

        issue_idx(row0, isets[0])
        issue_idx(row0 + 1, isets[1])
        wait_idx(isets[0])
        issue_gathers(row0, dsets[0], isets[0])
        issue_idx(row0 + 2, isets[2])

        def body(j, carry):
            g0 = row0 + 4 * j
            for p in range(4):
                wait_gathers(dsets[p % 2], isets[p])
                wait_idx(isets[(p + 1) % 4])
                issue_gathers(jnp.minimum(g0 + p + 1, last),
                              dsets[(p + 1) % 2], isets[(p + 1) % 4])
                issue_idx(jnp.minimum(g0 + p + 3, last), isets[(p + 3) % 4])
                compute(dsets[p % 2], isets[p])
            return carry

        lax.fori_loop(0, gps // 4, body, 0)
        wait_gathers(dsets[0], isets[0])
        wait_idx(isets[1])
        wait_idx(isets[2])
        plsc.subcore_barrier()
        pltpu.sync_copy(out_sh.at[pl.ds(s * rpw, rpw)],
                        outp_hbm.at[c, pl.ds(s * rpw, rpw)])

    return kern(ei3, p3d, h, zD)


def kernel(x, edge_index, W, att_src, att_dst, bias):
    N, D = x.shape
    E = edge_index.shape[1]
    H, C = att_src.shape

    eye = jnp.eye(H, dtype=jnp.float32)
    Asrc = (att_src[:, :, None] * eye[:, None, :]).reshape(H * C, H)
    Adst = (att_dst[:, :, None] * eye[:, None, :]).reshape(H * C, H)
    A2 = jnp.concatenate([Asrc, Asrc, Adst, Adst], axis=1)

    NP = ((N + 2047) // 2048) * 2048
    xp = jnp.concatenate([x, jnp.zeros((NP - N, D), jnp.float32)], axis=0)
    h, t = _tc_prep(xp, W, A2, BN=1024)
    s_tab = t[:, :16]
    d_tab = t[:, 16:]

    NG = ((E + NW * G * 4 - 1) // (NW * G * 4)) * NW * G * 4
    pad1 = (N + jnp.arange(NG - E, dtype=jnp.int32) % (NP - N))
    pad = jnp.stack([pad1, pad1])
    ei3 = (jnp.concatenate([edge_index, pad], axis=1)
           .reshape(2, NG // G, G).transpose(1, 0, 2))
    z16 = jnp.zeros((NP, 16), jnp.float32)
    zD = jnp.zeros((NP, D), jnp.float32)

    p3d, dens = _edge_pass_a(ei3, s_tab, d_tab, z16)
    parts = _edge_pass_b(ei3, p3d, h, zD)

    l = jnp.arange(D)
    Bsel = (jnp.arange(16)[:, None] == (l[None, :] // C)).astype(jnp.float32)
    out = _tc_finish(parts, dens, Bsel, bias.reshape(1, D), BN=1024)
    return out[:N]
